# Initial kernel scaffold; baseline (speedup 1.0000x reference)
import jax
import jax.numpy as jnp
from jax.experimental import pallas as pl


def kernel(x, edge_index, edge_weights, W, att_src, att_dst):
    raise NotImplementedError("write your pallas kernel here")


# speedup vs baseline: 35.8565x; 35.8565x over previous
import functools

import jax
import jax.numpy as jnp
import numpy as np
from jax import lax
from jax.experimental import pallas as pl
from jax.experimental.pallas import tpu as pltpu
from jax.experimental.pallas import tpu_sc as plsc

N = 10000
F = 128
H = 4
D = 32
E = 320000
NEG = 0.1

NC = 2
NS = 16
NW = NC * NS
K = 128
CPW = 80
EPW = K * CPW
EPAD = EPW * NW
NROW = 10240
RPT = NROW // NS
AW = 16

_f32 = jnp.float32
_i32 = jnp.int32


def _proj_body(x_ref, w_ref, ma_ref, ms_ref, h_ref, acat_ref, sself_ref):
    h = jnp.dot(x_ref[...], w_ref[...], preferred_element_type=_f32)
    h_ref[...] = h
    acat_ref[...] = jnp.dot(h, ma_ref[...], preferred_element_type=_f32)
    z = jnp.dot(h, ms_ref[...], preferred_element_type=_f32)
    sself_ref[...] = jnp.exp(jnp.maximum(z, NEG * z))


def _proj(x, W, M_att, M_sum):
    bn = 1000
    return pl.pallas_call(
        _proj_body,
        grid=(N // bn,),
        in_specs=[
            pl.BlockSpec((bn, F), lambda i: (i, 0)),
            pl.BlockSpec((F, F), lambda i: (0, 0)),
            pl.BlockSpec((F, AW), lambda i: (0, 0)),
            pl.BlockSpec((F, H), lambda i: (0, 0)),
        ],
        out_specs=[
            pl.BlockSpec((bn, F), lambda i: (i, 0)),
            pl.BlockSpec((bn, AW), lambda i: (i, 0)),
            pl.BlockSpec((bn, H), lambda i: (i, 0)),
        ],
        out_shape=[
            jax.ShapeDtypeStruct((N, F), _f32),
            jax.ShapeDtypeStruct((N, AW), _f32),
            jax.ShapeDtypeStruct((N, H), _f32),
        ],
    )(x, W, M_att, M_sum)


def _edge_body(acat_hbm, src_hbm, dst_hbm, ew_hbm, h_hbm, outm_hbm, outd_hbm,

               src_v, dst_v, ew_v, as_v, ad_v, rows_v, sden_v,
               accm_sh, accd_sh, sem_g, sem_a):
    c = lax.axis_index("c")
    tid = lax.axis_index("s")
    wid = tid * NC + c

    zero16 = jnp.zeros((16,), _f32)

    def _zero_row(e, carry):
        for q in range(F // 16):
            rows_v[e, pl.ds(16 * q, 16)] = zero16
        sden_v[e, pl.ds(0, 16)] = zero16
        return carry

    lax.fori_loop(0, K, _zero_row, 0)
    for b in range(RPT // K):
        r = tid * RPT + b * K
        pltpu.sync_copy(rows_v, accm_sh.at[pl.ds(r, K), :])
        pltpu.sync_copy(sden_v, accd_sh.at[pl.ds(r, K), :])
    plsc.subcore_barrier()

    iota16 = lax.iota(_i32, 16)

    def _chunk(ci, carry):
        base = wid * EPW + ci * K
        pltpu.sync_copy(src_hbm.at[pl.ds(base, K)], src_v)
        pltpu.sync_copy(dst_hbm.at[pl.ds(base, K)], dst_v)
        pltpu.sync_copy(ew_hbm.at[pl.ds(base, K)], ew_v)
        ga = pltpu.async_copy(acat_hbm.at[src_v], as_v, sem_a)
        gb = pltpu.async_copy(acat_hbm.at[dst_v], ad_v, sem_a)
        gather = pltpu.async_copy(h_hbm.at[src_v], rows_v, sem_g)
        ga.wait()
        gb.wait()

        for g in range(K // 16):
            wv = ew_v[pl.ds(16 * g, 16)]
            row16 = iota16 + (16 * g)
            for hh in range(H):
                a = plsc.load_gather(as_v, [row16, jnp.full((16,), hh, _i32)])
                b = plsc.load_gather(
                    ad_v, [row16, jnp.full((16,), H + hh, _i32)])
                e = a + b
                e = jnp.maximum(e, NEG * e)
                s = jnp.exp(e) * wv
                plsc.store_scatter(
                    sden_v, [row16, jnp.full((16,), hh, _i32)], s)

        gather.wait()

        def _scale(e, carry2):
            sv = sden_v[e, pl.ds(0, 16)]
            for hh in range(H):
                sc = sv[hh]
                for q2 in range(2):
                    q = hh * 2 + q2
                    rows_v[e, pl.ds(16 * q, 16)] = (
                        rows_v[e, pl.ds(16 * q, 16)] * sc)
            return carry2

        lax.fori_loop(0, K, _scale, 0)

        pltpu.sync_copy(rows_v, accm_sh.at[dst_v], add=True)
        pltpu.sync_copy(sden_v, accd_sh.at[dst_v], add=True)
        return carry

    lax.fori_loop(0, CPW, _chunk, 0)
    plsc.subcore_barrier()

    for b in range(RPT // K):
        r = tid * RPT + b * K
        pltpu.sync_copy(accm_sh.at[pl.ds(r, K), :], rows_v)
        pltpu.sync_copy(rows_v, outm_hbm.at[c, pl.ds(r, K), :])
        pltpu.sync_copy(accd_sh.at[pl.ds(r, K), :], sden_v)
        pltpu.sync_copy(sden_v, outd_hbm.at[c, pl.ds(r, K), :])


_edge_call = functools.partial(
    pl.kernel,
    out_type=[
        jax.ShapeDtypeStruct((NC, NROW, F), _f32),
        jax.ShapeDtypeStruct((NC, NROW, AW), _f32),
    ],
    mesh=plsc.VectorSubcoreMesh(core_axis_name="c", subcore_axis_name="s"),
    compiler_params=pltpu.CompilerParams(
        needs_layout_passes=False, use_tc_tiling_on_sc=False),
    scratch_types=[
        pltpu.VMEM((K,), _i32),
        pltpu.VMEM((K,), _i32),
        pltpu.VMEM((K,), _f32),
        pltpu.VMEM((K, AW), _f32),
        pltpu.VMEM((K, AW), _f32),
        pltpu.VMEM((K, F), _f32),
        pltpu.VMEM((K, AW), _f32),
        pltpu.VMEM_SHARED((NROW, F), _f32),
        pltpu.VMEM_SHARED((NROW, AW), _f32),
        pltpu.SemaphoreType.DMA,
        pltpu.SemaphoreType.DMA,
    ],
)(_edge_body)


def _combine_body(m0_ref, m1_ref, d0_ref, d1_ref, h_ref, ss_ref, r_ref,
                  out_ref):
    num = m0_ref[...] + m1_ref[...] + h_ref[...] * jnp.dot(
        ss_ref[...], r_ref[...], preferred_element_type=_f32)
    den = jnp.dot(d0_ref[...] + d1_ref[...] + ss_ref[...], r_ref[...],
                  preferred_element_type=_f32)
    out_ref[...] = num / (den + 1e-16)


def _combine(m0, m1, d0, d1, h, sself, R):
    bn = 1000
    return pl.pallas_call(
        _combine_body,
        grid=(N // bn,),
        in_specs=[
            pl.BlockSpec((bn, F), lambda i: (i, 0)),
            pl.BlockSpec((bn, F), lambda i: (i, 0)),
            pl.BlockSpec((bn, H), lambda i: (i, 0)),
            pl.BlockSpec((bn, H), lambda i: (i, 0)),
            pl.BlockSpec((bn, F), lambda i: (i, 0)),
            pl.BlockSpec((bn, H), lambda i: (i, 0)),
            pl.BlockSpec((H, F), lambda i: (0, 0)),
        ],
        out_specs=pl.BlockSpec((bn, F), lambda i: (i, 0)),
        out_shape=jax.ShapeDtypeStruct((N, F), _f32),
    )(m0, m1, d0, d1, h, sself, R)


def kernel(x, edge_index, edge_weights, W, att_src, att_dst):
    eyeH = jnp.eye(H, dtype=_f32)
    M_src = (att_src[:, :, None] * eyeH[:, None, :]).reshape(F, H)
    M_dst = (att_dst[:, :, None] * eyeH[:, None, :]).reshape(F, H)
    M_att = jnp.concatenate(
        [M_src, M_dst, jnp.zeros((F, AW - 2 * H), _f32)], axis=1)
    M_sum = M_src + M_dst

    h, acat, sself = _proj(x, W, M_att, M_sum)

    pad = EPAD - E
    src_p = jnp.concatenate([edge_index[0], jnp.zeros((pad,), _i32)])
    dst_p = jnp.concatenate([edge_index[1], jnp.full((pad,), N, _i32)])
    ew_p = jnp.concatenate([edge_weights, jnp.zeros((pad,), _f32)])

    acat_p = jnp.concatenate([acat, jnp.zeros((NROW - N, AW), _f32)])
    msg, den = _edge_call(acat_p, src_p, dst_p, ew_p, h)

    R = jnp.asarray(np.kron(np.eye(H), np.ones((1, D))), dtype=_f32)
    return _combine(msg[0, :N], msg[1, :N],
                    den[0, :N, :H], den[1, :N, :H],
                    h, sself, R)


# speedup vs baseline: 43.3220x; 1.2082x over previous
import functools

import jax
import jax.numpy as jnp
import numpy as np
from jax import lax
from jax.experimental import pallas as pl
from jax.experimental.pallas import tpu as pltpu
from jax.experimental.pallas import tpu_sc as plsc

N = 10000
F = 128
H = 4
D = 32
E = 320000
NEG = 0.1

NC = 2
NS = 16
NW = NC * NS
K = 128
CPW = 80
EPW = K * CPW
EPAD = EPW * NW
NCH = EPAD // K
RPT = N // NS
AW = 16
DW = 8

_f32 = jnp.float32
_i32 = jnp.int32


def _proj_body(x_ref, w_ref, ma_ref, ms_ref, h_ref, acat_ref, sself_ref):
    h = jnp.dot(x_ref[...], w_ref[...], preferred_element_type=_f32)
    h_ref[...] = h
    acat_ref[...] = jnp.dot(h, ma_ref[...], preferred_element_type=_f32)
    z = jnp.dot(h, ms_ref[...], preferred_element_type=_f32)
    sself_ref[...] = jnp.exp(jnp.maximum(z, NEG * z))


def _proj(x, W, M_att, M_sum):
    bn = 1000
    return pl.pallas_call(
        _proj_body,
        grid=(N // bn,),
        in_specs=[
            pl.BlockSpec((bn, F), lambda i: (i, 0)),
            pl.BlockSpec((F, F), lambda i: (0, 0)),
            pl.BlockSpec((F, AW), lambda i: (0, 0)),
            pl.BlockSpec((F, H), lambda i: (0, 0)),
        ],
        out_specs=[
            pl.BlockSpec((bn, F), lambda i: (i, 0)),
            pl.BlockSpec((bn, AW), lambda i: (i, 0)),
            pl.BlockSpec((bn, H), lambda i: (i, 0)),
        ],
        out_shape=[
            jax.ShapeDtypeStruct((N, F), _f32),
            jax.ShapeDtypeStruct((N, AW), _f32),
            jax.ShapeDtypeStruct((N, H), _f32),
        ],
    )(x, W, M_att, M_sum)


def _edge_body(acat_hbm, sdew_hbm, h_hbm, z8_hbm, outm_hbm, outd_hbm,
               sdew0, sdew1, as_v, ad_v, rows0, rows1, sden0, sden1, sc2_v,
               accm_sh, accd_sh, sem_g, sem_a, sem_s):
    c = lax.axis_index("c")
    tid = lax.axis_index("s")
    wid = tid * NC + c
    sdew_b = (sdew0, sdew1)
    rows_b = (rows0, rows1)
    sden_b = (sden0, sden1)

    zero16 = jnp.zeros((16,), _f32)
    zero16i = jnp.zeros((16,), _i32)
    iota16 = lax.iota(_i32, 16)

    for b in range(2):
        for p in range(3):
            for q in range(K // 16):
                sdew_b[b][p, pl.ds(16 * q, 16)] = zero16i
        pltpu.sync_copy(z8_hbm, sden_b[b])

    def _zero_row(e, carry):
        for q in range(F // 16):
            rows0[e, pl.ds(16 * q, 16)] = zero16
            rows1[e, pl.ds(16 * q, 16)] = zero16
        return carry

    lax.fori_loop(0, K, _zero_row, 0)

    for off, cn in ((0, 128), (128, 128), (256, 128), (384, 128), (512, 113)):
        r = tid * RPT + off
        pltpu.sync_copy(rows0.at[pl.ds(0, cn), :], accm_sh.at[pl.ds(r, cn), :])
        pltpu.sync_copy(sden0.at[pl.ds(0, cn), :], accd_sh.at[pl.ds(r, cn), :])
    plsc.subcore_barrier()

    for b in range(2):
        pltpu.async_copy(rows_b[b], accm_sh.at[sdew_b[b].at[1]], sem_s,
                         add=True)
        pltpu.async_copy(sden_b[b], accd_sh.at[sdew_b[b].at[1]], sem_s,
                         add=True)

    def _outer(oi, carry):
        for b in range(2):
            ci = oi * 2 + b
            pltpu.make_async_copy(
                rows_b[b], accm_sh.at[sdew_b[b].at[1]], sem_s).wait()
            pltpu.make_async_copy(
                sden_b[b], accd_sh.at[sdew_b[b].at[1]], sem_s).wait()

            j = wid * CPW + ci
            pltpu.sync_copy(sdew_hbm.at[:, j, :], sdew_b[b])
            g_rows = pltpu.async_copy(
                h_hbm.at[sdew_b[b].at[0]], rows_b[b], sem_g)
            g_as = pltpu.async_copy(
                acat_hbm.at[sdew_b[b].at[0]], as_v, sem_a)
            g_ad = pltpu.async_copy(
                acat_hbm.at[sdew_b[b].at[1]], ad_v, sem_a)
            g_as.wait()
            g_ad.wait()

            for g in range(K // 16):
                ew = plsc.bitcast(sdew_b[b][2, pl.ds(16 * g, 16)], _f32)
                row16 = iota16 + (16 * g)
                for hh in range(H):
                    a = plsc.load_gather(
                        as_v, [row16, jnp.full((16,), hh, _i32)])
                    d = plsc.load_gather(
                        ad_v, [row16, jnp.full((16,), H + hh, _i32)])
                    e = a + d
                    e = jnp.maximum(e, NEG * e)
                    s = jnp.exp(e) * ew
                    col = jnp.full((16,), hh, _i32)
                    plsc.store_scatter(sc2_v, [row16, col], s)
                    plsc.store_scatter(sden_b[b], [row16, col], s)

            g_rows.wait()

            def _scale(e, carry2, _rows=rows_b[b]):
                sv = sc2_v[e, pl.ds(0, 16)]
                for hh in range(H):
                    sc = sv[hh]
                    for q2 in range(2):
                        q = hh * 2 + q2
                        _rows[e, pl.ds(16 * q, 16)] = (
                            _rows[e, pl.ds(16 * q, 16)] * sc)
                return carry2

            lax.fori_loop(0, K, _scale, 0)

            pltpu.async_copy(rows_b[b], accm_sh.at[sdew_b[b].at[1]], sem_s,
                             add=True)
            pltpu.async_copy(sden_b[b], accd_sh.at[sdew_b[b].at[1]], sem_s,
                             add=True)
        return carry

    lax.fori_loop(0, CPW // 2, _outer, 0)

    for b in range(2):
        pltpu.make_async_copy(
            rows_b[b], accm_sh.at[sdew_b[b].at[1]], sem_s).wait()
        pltpu.make_async_copy(
            sden_b[b], accd_sh.at[sdew_b[b].at[1]], sem_s).wait()
    plsc.subcore_barrier()

    for off, cn in ((0, 128), (128, 128), (256, 128), (384, 128), (512, 113)):
        r = tid * RPT + off
        pltpu.sync_copy(accm_sh.at[pl.ds(r, cn), :], rows0.at[pl.ds(0, cn), :])
        pltpu.sync_copy(rows0.at[pl.ds(0, cn), :],
                        outm_hbm.at[c, pl.ds(r, cn), :])
        pltpu.sync_copy(accd_sh.at[pl.ds(r, cn), :], sden0.at[pl.ds(0, cn), :])
        pltpu.sync_copy(sden0.at[pl.ds(0, cn), :],
                        outd_hbm.at[c, pl.ds(r, cn), :])


_edge_call = functools.partial(
    pl.kernel,
    out_type=[
        jax.ShapeDtypeStruct((NC, N, F), _f32),
        jax.ShapeDtypeStruct((NC, N, DW), _f32),
    ],
    mesh=plsc.VectorSubcoreMesh(core_axis_name="c", subcore_axis_name="s"),
    compiler_params=pltpu.CompilerParams(
        needs_layout_passes=False, use_tc_tiling_on_sc=False),
    scratch_types=[
        pltpu.VMEM((3, K), _i32),
        pltpu.VMEM((3, K), _i32),
        pltpu.VMEM((K, AW), _f32),
        pltpu.VMEM((K, AW), _f32),
        pltpu.VMEM((K, F), _f32),
        pltpu.VMEM((K, F), _f32),
        pltpu.VMEM((K, DW), _f32),
        pltpu.VMEM((K, DW), _f32),
        pltpu.VMEM((K, AW), _f32),
        pltpu.VMEM_SHARED((N, F), _f32),
        pltpu.VMEM_SHARED((N, DW), _f32),
        pltpu.SemaphoreType.DMA,
        pltpu.SemaphoreType.DMA,
        pltpu.SemaphoreType.DMA,
    ],
)(_edge_body)


def _combine_body(m0_ref, m1_ref, d0_ref, d1_ref, h_ref, ss_ref, r_ref,
                  out_ref):
    num = m0_ref[...] + m1_ref[...] + h_ref[...] * jnp.dot(
        ss_ref[...], r_ref[...], preferred_element_type=_f32)
    den = jnp.dot(d0_ref[...] + d1_ref[...] + ss_ref[...], r_ref[...],
                  preferred_element_type=_f32)
    out_ref[...] = num / (den + 1e-16)


def _combine(m0, m1, d0, d1, h, sself, R):
    bn = 1000
    return pl.pallas_call(
        _combine_body,
        grid=(N // bn,),
        in_specs=[
            pl.BlockSpec((bn, F), lambda i: (i, 0)),
            pl.BlockSpec((bn, F), lambda i: (i, 0)),
            pl.BlockSpec((bn, H), lambda i: (i, 0)),
            pl.BlockSpec((bn, H), lambda i: (i, 0)),
            pl.BlockSpec((bn, F), lambda i: (i, 0)),
            pl.BlockSpec((bn, H), lambda i: (i, 0)),
            pl.BlockSpec((H, F), lambda i: (0, 0)),
        ],
        out_specs=pl.BlockSpec((bn, F), lambda i: (i, 0)),
        out_shape=jax.ShapeDtypeStruct((N, F), _f32),
    )(m0, m1, d0, d1, h, sself, R)


def kernel(x, edge_index, edge_weights, W, att_src, att_dst):
    eyeH = jnp.eye(H, dtype=_f32)
    M_src = (att_src[:, :, None] * eyeH[:, None, :]).reshape(F, H)
    M_dst = (att_dst[:, :, None] * eyeH[:, None, :]).reshape(F, H)
    M_att = jnp.concatenate(
        [M_src, M_dst, jnp.zeros((F, AW - 2 * H), _f32)], axis=1)
    M_sum = M_src + M_dst

    h, acat, sself = _proj(x, W, M_att, M_sum)

    pad = EPAD - E
    src_p = jnp.concatenate([edge_index[0], jnp.zeros((pad,), _i32)])
    dst_p = jnp.concatenate([edge_index[1], jnp.zeros((pad,), _i32)])
    ew_p = jnp.concatenate([edge_weights, jnp.zeros((pad,), _f32)])
    sdew = jnp.stack([
        src_p.reshape(NCH, K),
        dst_p.reshape(NCH, K),
        lax.bitcast_convert_type(ew_p, _i32).reshape(NCH, K),
    ])
    z8 = jnp.zeros((K, DW), _f32)

    msg, den = _edge_call(acat, sdew, h, z8)

    R = jnp.asarray(np.kron(np.eye(H), np.ones((1, D))), dtype=_f32)
    return _combine(msg[0], msg[1],
                    den[0, :, :H], den[1, :, :H],
                    h, sself, R)
